# baseline (speedup 1.0000x reference)
import functools

import jax
import jax.numpy as jnp
from jax import lax
from jax.experimental import pallas as pl
from jax.experimental.pallas import tpu as pltpu
from jax.experimental.pallas import tpu_sc as plsc

N = 10000
E = 320000
D_IN = 128
D_H = 256
D_OUT = 128

NC = 2
NS = 16
NW = NC * NS
CHUNK = 128
NCHUNK = 80
EPT = NCHUNK * CHUNK
EPAD = NW * EPT - E
N2 = N + 16
GRP = 8
NGROUP = NCHUNK // GRP
RPS = N2 // NS

_mesh = plsc.VectorSubcoreMesh(core_axis_name="c", subcore_axis_name="s")


def _make_sc_segsum(with_deg: bool):
    out_type = [jax.ShapeDtypeStruct((NC, N2, 128), jnp.float32)]
    scratch = (
        [pltpu.VMEM((GRP, CHUNK), jnp.int32) for _ in range(2)]
        + [pltpu.VMEM((GRP, CHUNK), jnp.int32)]
        + [pltpu.VMEM((CHUNK, 128), jnp.float32) for _ in range(2)]
        + [pltpu.VMEM_SHARED((N2, 128), jnp.float32)]
        + [pltpu.SemaphoreType.DMA for _ in range(2)]
    )
    if with_deg:
        out_type.append(jax.ShapeDtypeStruct((NC, N2, 16), jnp.float32))
        scratch += (
            [pltpu.VMEM((CHUNK, 16), jnp.float32)]
            + [pltpu.VMEM_SHARED((N2, 16), jnp.float32)]
        )

    def body(table, edges, zrows, *rest):
        if with_deg:
            (zdeg, ones_h, p_out, degp_out, idxs0, idxs1, idx_d, rows0,
             rows1, acc, sem0, sem1, ones_v, accd) = rest
        else:
            (p_out, idxs0, idxs1, idx_d, rows0, rows1, acc, sem0,
             sem1) = rest
        idx_s = (idxs0, idxs1)
        c = lax.axis_index("c")
        s = lax.axis_index("s")
        wid = s * NC + c
        r0 = s * RPS
        pltpu.sync_copy(zrows.at[pl.ds(r0, RPS)], acc.at[pl.ds(r0, RPS)])
        if with_deg:
            pltpu.sync_copy(zdeg.at[pl.ds(r0, RPS)], accd.at[pl.ds(r0, RPS)])
            pltpu.sync_copy(ones_h, ones_v)
        pltpu.sync_copy(edges.at[0, wid, pl.ds(0, GRP)], idxs0)
        plsc.subcore_barrier()

        def both_parities(par, fn):
            @pl.when(par == 0)
            def _():
                fn(0)

            @pl.when(par == 1)
            def _():
                fn(1)

        def gather(j, buf, sem):
            def issue(par):
                pltpu.async_copy(table.at[idx_s[par].at[j % GRP]], buf, sem)

            both_parities((j // GRP) % 2, issue)

        def drain_scatter(jm, buf, sem):
            pltpu.make_async_copy(table.at[idxs0.at[0]], buf, sem).wait()
            pltpu.sync_copy(buf, acc.at[idx_d.at[jm]], add=True)
            if with_deg:
                pltpu.sync_copy(ones_v, accd.at[idx_d.at[jm]], add=True)

        gather(0, rows0, sem0)

        def step(t, carry):
            j0 = 2 * t
            g = j0 // GRP
            jm = j0 % GRP

            @pl.when(jm == 0)
            def _():
                pltpu.sync_copy(edges.at[1, wid, pl.ds(g * GRP, GRP)], idx_d)

            gather(j0 + 1, rows1, sem1)
            drain_scatter(jm, rows0, sem0)

            @pl.when(jnp.logical_and(jm == GRP // 4, g + 1 < NGROUP))
            def _():
                def load(par):
                    pltpu.sync_copy(
                        edges.at[0, wid, pl.ds((g + 1) * GRP, GRP)],
                        idx_s[par])

                both_parities((g + 1) % 2, load)

            @pl.when(t + 1 < NCHUNK // 2)
            def _():
                gather(j0 + 2, rows0, sem0)

            drain_scatter(jm + 1, rows1, sem1)
            return carry

        lax.fori_loop(0, NCHUNK // 2, step, 0)
        plsc.subcore_barrier()
        pltpu.sync_copy(acc.at[pl.ds(r0, RPS)], p_out.at[c, pl.ds(r0, RPS)])
        if with_deg:
            pltpu.sync_copy(accd.at[pl.ds(r0, RPS)],
                            degp_out.at[c, pl.ds(r0, RPS)])

    return pl.kernel(body, out_type=out_type, mesh=_mesh,
                     scratch_types=scratch,
                     compiler_params=pltpu.CompilerParams(
                         use_tc_tiling_on_sc=False))


_sc_segsum_deg = _make_sc_segsum(with_deg=True)
_sc_segsum = _make_sc_segsum(with_deg=False)

BN = 1000
_GRID = N // BN


def _tc0_body(x, wr, bl, xr):
    xr[...] = (jnp.dot(x[...], wr[...], preferred_element_type=jnp.float32)
               + bl[...])


def _tc1_body(p, d, xr, wl, hpre, stats):
    i = pl.program_id(0)
    deg = jnp.maximum(d[0, :, 0:1] + d[1, :, 0:1], 1.0)
    agg = (p[0] + p[1]) / deg
    h = jnp.dot(agg, wl[...], preferred_element_type=jnp.float32) + xr[...]
    hpre[...] = h
    ss = jnp.concatenate([jnp.sum(h, 0, keepdims=True),
                          jnp.sum(h * h, 0, keepdims=True)], axis=0)

    @pl.when(i == 0)
    def _():
        stats[...] = ss

    @pl.when(i != 0)
    def _():
        stats[...] = stats[...] + ss


def _bn_relu(hpre, stats, gamma, beta):
    st = stats[...]
    mean = st[0:1, :] * (1.0 / N)
    var = st[1:2, :] * (1.0 / N) - mean * mean
    scale = gamma[...] * lax.rsqrt(var + 1e-5)
    return jnp.maximum((hpre[...] - mean) * scale + beta[...], 0.0)


def _tc2a_body(hpre, stats, gamma, beta, wl2, p2):
    h = _bn_relu(hpre, stats, gamma, beta)
    p2[...] = jnp.dot(h, wl2[...], preferred_element_type=jnp.float32)


def _tc2b_body(hpre, stats, gamma, beta, wr2, bl2, r2b):
    h = _bn_relu(hpre, stats, gamma, beta)
    r2b[...] = (jnp.dot(h, wr2[...], preferred_element_type=jnp.float32)
                + bl2[...])


def _tc3_body(q, d, r2b, out):
    deg = jnp.maximum(d[0, :, 0:1] + d[1, :, 0:1], 1.0)
    out[...] = (q[0] + q[1]) / deg + r2b[...]


def _row_spec(w):
    return pl.BlockSpec((BN, w), lambda i: (i, 0))


def _pair_spec(w):
    return pl.BlockSpec((2, BN, w), lambda i: (0, i, 0))


def _full_spec(shape):
    return pl.BlockSpec(shape, lambda i: tuple(0 for _ in shape))


_tc0 = pl.pallas_call(
    _tc0_body,
    grid=(_GRID,),
    in_specs=[_row_spec(128), _full_spec((128, 256)), _full_spec((1, 256))],
    out_specs=_row_spec(256),
    out_shape=jax.ShapeDtypeStruct((N, 256), jnp.float32),
)

_tc1 = pl.pallas_call(
    _tc1_body,
    grid=(_GRID,),
    in_specs=[_pair_spec(128), _pair_spec(16),
              _row_spec(256), _full_spec((128, 256))],
    out_specs=[_row_spec(256), _full_spec((2, 256))],
    out_shape=[jax.ShapeDtypeStruct((N, 256), jnp.float32),
               jax.ShapeDtypeStruct((2, 256), jnp.float32)],
)

_tc2a = pl.pallas_call(
    _tc2a_body,
    grid=(_GRID,),
    in_specs=[_row_spec(256), _full_spec((2, 256)), _full_spec((1, 256)),
              _full_spec((1, 256)), _full_spec((256, 128))],
    out_specs=_row_spec(128),
    out_shape=jax.ShapeDtypeStruct((N, 128), jnp.float32),
)

_tc2b = pl.pallas_call(
    _tc2b_body,
    grid=(_GRID,),
    in_specs=[_row_spec(256), _full_spec((2, 256)), _full_spec((1, 256)),
              _full_spec((1, 256)), _full_spec((256, 128)),
              _full_spec((1, 128))],
    out_specs=_row_spec(128),
    out_shape=jax.ShapeDtypeStruct((N, 128), jnp.float32),
)

_tc3 = pl.pallas_call(
    _tc3_body,
    grid=(_GRID,),
    in_specs=[_pair_spec(128), _pair_spec(16), _row_spec(128)],
    out_specs=_row_spec(128),
    out_shape=jax.ShapeDtypeStruct((N, 128), jnp.float32),
)


def kernel(x, edge_index, Wl1, bl1, Wr1, Wl2, bl2, Wr2, gamma, beta):
    filler = jnp.stack([jnp.zeros((EPAD,), jnp.int32),
                        jnp.full((EPAD,), N, jnp.int32)])
    edges = jnp.concatenate([edge_index, filler],
                            axis=1).reshape(2, NW, NCHUNK, CHUNK)
    zrows = jnp.zeros((N2, 128), jnp.float32)
    zdeg = jnp.zeros((N2, 16), jnp.float32)
    ones_h = jnp.ones((CHUNK, 16), jnp.float32)

    xr = _tc0(x, Wr1, bl1.reshape(1, 256))
    P, degP = _sc_segsum_deg(x, edges, zrows, zdeg, ones_h)
    hpre, stats = _tc1(P, degP, xr, Wl1)
    gam = gamma.reshape(1, 256)
    bet = beta.reshape(1, 256)
    p2 = _tc2a(hpre, stats, gam, bet, Wl2)
    r2b = _tc2b(hpre, stats, gam, bet, Wr2, bl2.reshape(1, 128))
    (Q,) = _sc_segsum(p2, edges, zrows)
    out = _tc3(Q, degP, r2b)
    return out


# speedup vs baseline: 3.2702x; 3.2702x over previous
import functools

import jax
import jax.numpy as jnp
from jax import lax
from jax.experimental import pallas as pl
from jax.experimental.pallas import tpu as pltpu
from jax.experimental.pallas import tpu_sc as plsc

N = 10000
E = 320000
D_IN = 128
D_H = 256
D_OUT = 128

NC = 2
NS = 16
NW = NC * NS
EPT = E // NW
CHUNK = 125
NCHUNK = EPT // CHUNK
GRP = 16
NGROUP = NCHUNK // GRP
RPS = N // NS

_mesh = plsc.VectorSubcoreMesh(core_axis_name="c", subcore_axis_name="s")


def _make_sc_segsum(with_deg: bool):
    out_type = [jax.ShapeDtypeStruct((NC, N, 128), jnp.float32)]
    scratch = (
        [pltpu.VMEM((GRP, CHUNK), jnp.int32) for _ in range(2)]
        + [pltpu.VMEM((GRP, CHUNK), jnp.int32)]
        + [pltpu.VMEM((CHUNK, 128), jnp.float32) for _ in range(2)]
        + [pltpu.VMEM_SHARED((N, 128), jnp.float32)]
        + [pltpu.SemaphoreType.DMA for _ in range(2)]
    )
    if with_deg:
        out_type.append(jax.ShapeDtypeStruct((NC, N, 16), jnp.float32))
        scratch += (
            [pltpu.VMEM((CHUNK, 16), jnp.float32)]
            + [pltpu.VMEM_SHARED((N, 16), jnp.float32)]
        )

    def body(table, srcr, dstr, zrows, *rest):
        if with_deg:
            (zdeg, ones_h, p_out, degp_out, idxs0, idxs1, idx_d, rows0,
             rows1, acc, sem0, sem1, ones_v, accd) = rest
        else:
            (p_out, idxs0, idxs1, idx_d, rows0, rows1, acc, sem0,
             sem1) = rest
        idx_s = (idxs0, idxs1)
        c = lax.axis_index("c")
        s = lax.axis_index("s")
        wid = s * NC + c
        r0 = s * RPS
        pltpu.sync_copy(zrows.at[pl.ds(r0, RPS)], acc.at[pl.ds(r0, RPS)])
        if with_deg:
            pltpu.sync_copy(zdeg.at[pl.ds(r0, RPS)], accd.at[pl.ds(r0, RPS)])
            pltpu.sync_copy(ones_h, ones_v)
        pltpu.sync_copy(srcr.at[wid, pl.ds(0, GRP)], idxs0)
        plsc.subcore_barrier()

        def both_parities(par, fn):
            @pl.when(par == 0)
            def _():
                fn(0)

            @pl.when(par == 1)
            def _():
                fn(1)

        def gather(j, buf, sem):
            def issue(par):
                pltpu.async_copy(table.at[idx_s[par].at[j % GRP]], buf, sem)

            both_parities((j // GRP) % 2, issue)

        def drain_scatter(jm, buf, sem):
            pltpu.make_async_copy(table.at[idxs0.at[0]], buf, sem).wait()
            pltpu.sync_copy(buf, acc.at[idx_d.at[jm]], add=True)
            if with_deg:
                pltpu.sync_copy(ones_v, accd.at[idx_d.at[jm]], add=True)

        gather(0, rows0, sem0)

        def step(t, carry):
            j0 = 2 * t
            g = j0 // GRP
            jm = j0 % GRP

            @pl.when(jm == 0)
            def _():
                pltpu.sync_copy(dstr.at[wid, pl.ds(g * GRP, GRP)], idx_d)

            gather(j0 + 1, rows1, sem1)
            drain_scatter(jm, rows0, sem0)

            @pl.when(jnp.logical_and(jm == GRP // 4, g + 1 < NGROUP))
            def _():
                def load(par):
                    pltpu.sync_copy(srcr.at[wid, pl.ds((g + 1) * GRP, GRP)],
                                    idx_s[par])

                both_parities((g + 1) % 2, load)

            @pl.when(t + 1 < NCHUNK // 2)
            def _():
                gather(j0 + 2, rows0, sem0)

            drain_scatter(jm + 1, rows1, sem1)
            return carry

        lax.fori_loop(0, NCHUNK // 2, step, 0)
        plsc.subcore_barrier()
        pltpu.sync_copy(acc.at[pl.ds(r0, RPS)], p_out.at[c, pl.ds(r0, RPS)])
        if with_deg:
            pltpu.sync_copy(accd.at[pl.ds(r0, RPS)],
                            degp_out.at[c, pl.ds(r0, RPS)])

    return pl.kernel(body, out_type=out_type, mesh=_mesh,
                     scratch_types=scratch,
                     compiler_params=pltpu.CompilerParams(
                         use_tc_tiling_on_sc=False))


_sc_segsum_deg = _make_sc_segsum(with_deg=True)
_sc_segsum = _make_sc_segsum(with_deg=False)

BN = 1000
_GRID = N // BN


def _tc0_body(x, wr, bl, xr):
    xr[...] = (jnp.dot(x[...], wr[...], preferred_element_type=jnp.float32)
               + bl[...])


def _tc1_body(p, d, xr, wl, hpre, stats):
    i = pl.program_id(0)
    deg = jnp.maximum(d[0, :, 0:1] + d[1, :, 0:1], 1.0)
    agg = (p[0] + p[1]) / deg
    h = jnp.dot(agg, wl[...], preferred_element_type=jnp.float32) + xr[...]
    hpre[...] = h
    ss = jnp.concatenate([jnp.sum(h, 0, keepdims=True),
                          jnp.sum(h * h, 0, keepdims=True)], axis=0)

    @pl.when(i == 0)
    def _():
        stats[...] = ss

    @pl.when(i != 0)
    def _():
        stats[...] = stats[...] + ss


def _bn_relu(hpre, stats, gamma, beta):
    st = stats[...]
    mean = st[0:1, :] * (1.0 / N)
    var = st[1:2, :] * (1.0 / N) - mean * mean
    scale = gamma[...] * lax.rsqrt(var + 1e-5)
    return jnp.maximum((hpre[...] - mean) * scale + beta[...], 0.0)


def _tc2a_body(hpre, stats, gamma, beta, wl2, p2):
    h = _bn_relu(hpre, stats, gamma, beta)
    p2[...] = jnp.dot(h, wl2[...], preferred_element_type=jnp.float32)


def _tc2b_body(hpre, stats, gamma, beta, wr2, bl2, r2b):
    h = _bn_relu(hpre, stats, gamma, beta)
    r2b[...] = (jnp.dot(h, wr2[...], preferred_element_type=jnp.float32)
                + bl2[...])


def _tc3_body(q, d, r2b, out):
    deg = jnp.maximum(d[0, :, 0:1] + d[1, :, 0:1], 1.0)
    out[...] = (q[0] + q[1]) / deg + r2b[...]


def _row_spec(w):
    return pl.BlockSpec((BN, w), lambda i: (i, 0))


def _pair_spec(w):
    return pl.BlockSpec((2, BN, w), lambda i: (0, i, 0))


def _full_spec(shape):
    return pl.BlockSpec(shape, lambda i: tuple(0 for _ in shape))


_tc0 = pl.pallas_call(
    _tc0_body,
    grid=(_GRID,),
    in_specs=[_row_spec(128), _full_spec((128, 256)), _full_spec((1, 256))],
    out_specs=_row_spec(256),
    out_shape=jax.ShapeDtypeStruct((N, 256), jnp.float32),
)

_tc1 = pl.pallas_call(
    _tc1_body,
    grid=(_GRID,),
    in_specs=[_pair_spec(128), _pair_spec(16),
              _row_spec(256), _full_spec((128, 256))],
    out_specs=[_row_spec(256), _full_spec((2, 256))],
    out_shape=[jax.ShapeDtypeStruct((N, 256), jnp.float32),
               jax.ShapeDtypeStruct((2, 256), jnp.float32)],
)

_tc2a = pl.pallas_call(
    _tc2a_body,
    grid=(_GRID,),
    in_specs=[_row_spec(256), _full_spec((2, 256)), _full_spec((1, 256)),
              _full_spec((1, 256)), _full_spec((256, 128))],
    out_specs=_row_spec(128),
    out_shape=jax.ShapeDtypeStruct((N, 128), jnp.float32),
)

_tc2b = pl.pallas_call(
    _tc2b_body,
    grid=(_GRID,),
    in_specs=[_row_spec(256), _full_spec((2, 256)), _full_spec((1, 256)),
              _full_spec((1, 256)), _full_spec((256, 128)),
              _full_spec((1, 128))],
    out_specs=_row_spec(128),
    out_shape=jax.ShapeDtypeStruct((N, 128), jnp.float32),
)

_tc3 = pl.pallas_call(
    _tc3_body,
    grid=(_GRID,),
    in_specs=[_pair_spec(128), _pair_spec(16), _row_spec(128)],
    out_specs=_row_spec(128),
    out_shape=jax.ShapeDtypeStruct((N, 128), jnp.float32),
)


def kernel(x, edge_index, Wl1, bl1, Wr1, Wl2, bl2, Wr2, gamma, beta):
    src = edge_index[0].reshape(NW, NCHUNK, CHUNK)
    dst = edge_index[1].reshape(NW, NCHUNK, CHUNK)
    zrows = jnp.zeros((N, 128), jnp.float32)
    zdeg = jnp.zeros((N, 16), jnp.float32)
    ones_h = jnp.ones((CHUNK, 16), jnp.float32)

    xr = _tc0(x, Wr1, bl1.reshape(1, 256))
    P, degP = _sc_segsum_deg(x, src, dst, zrows, zdeg, ones_h)
    hpre, stats = _tc1(P, degP, xr, Wl1)
    gam = gamma.reshape(1, 256)
    bet = beta.reshape(1, 256)
    p2 = _tc2a(hpre, stats, gam, bet, Wl2)
    r2b = _tc2b(hpre, stats, gam, bet, Wr2, bl2.reshape(1, 128))
    (Q,) = _sc_segsum(p2, src, dst, zrows)
    out = _tc3(Q, degP, r2b)
    return out


# speedup vs baseline: 3.2934x; 1.0071x over previous
import functools

import jax
import jax.numpy as jnp
from jax import lax
from jax.experimental import pallas as pl
from jax.experimental.pallas import tpu as pltpu
from jax.experimental.pallas import tpu_sc as plsc

N = 10000
E = 320000
D_IN = 128
D_H = 256
D_OUT = 128

NC = 2
NS = 16
NW = NC * NS
EPT = E // NW
CHUNK = 125
NCHUNK = EPT // CHUNK
GRP = 16
NGROUP = NCHUNK // GRP
RPS = N // NS

_mesh = plsc.VectorSubcoreMesh(core_axis_name="c", subcore_axis_name="s")


def _make_sc_segsum(with_deg: bool):
    out_type = [jax.ShapeDtypeStruct((NC, N, 128), jnp.float32)]
    scratch = (
        [pltpu.VMEM((GRP, CHUNK), jnp.int32) for _ in range(2)]
        + [pltpu.VMEM((GRP, CHUNK), jnp.int32)]
        + [pltpu.VMEM((CHUNK, 128), jnp.float32) for _ in range(2)]
        + [pltpu.VMEM_SHARED((N, 128), jnp.float32)]
        + [pltpu.SemaphoreType.DMA for _ in range(2)]
    )
    if with_deg:
        out_type.append(jax.ShapeDtypeStruct((NC, N, 16), jnp.float32))
        scratch += (
            [pltpu.VMEM((CHUNK, 16), jnp.float32)]
            + [pltpu.VMEM_SHARED((N, 16), jnp.float32)]
        )

    def body(table, srcr, dstr, zrows, *rest):
        if with_deg:
            (zdeg, ones_h, p_out, degp_out, idxs0, idxs1, idx_d, rows0,
             rows1, acc, sem0, sem1, ones_v, accd) = rest
        else:
            (p_out, idxs0, idxs1, idx_d, rows0, rows1, acc, sem0,
             sem1) = rest
        idx_s = (idxs0, idxs1)
        c = lax.axis_index("c")
        s = lax.axis_index("s")
        wid = s * NC + c
        r0 = s * RPS
        pltpu.async_copy(zrows.at[pl.ds(r0, RPS)], acc.at[pl.ds(r0, RPS)],
                         sem0)
        pltpu.async_copy(srcr.at[wid, pl.ds(0, GRP)], idxs0, sem1)
        if with_deg:
            pltpu.sync_copy(zdeg.at[pl.ds(r0, RPS)], accd.at[pl.ds(r0, RPS)])
            pltpu.sync_copy(ones_h, ones_v)
        pltpu.make_async_copy(zrows.at[pl.ds(r0, RPS)],
                              acc.at[pl.ds(r0, RPS)], sem0).wait()
        pltpu.make_async_copy(srcr.at[wid, pl.ds(0, GRP)], idxs0,
                              sem1).wait()
        plsc.subcore_barrier()

        def both_parities(par, fn):
            @pl.when(par == 0)
            def _():
                fn(0)

            @pl.when(par == 1)
            def _():
                fn(1)

        def gather(j, buf, sem):
            def issue(par):
                pltpu.async_copy(table.at[idx_s[par].at[j % GRP]], buf, sem)

            both_parities((j // GRP) % 2, issue)

        def drain_scatter(jm, buf, sem):
            pltpu.make_async_copy(table.at[idxs0.at[0]], buf, sem).wait()
            pltpu.sync_copy(buf, acc.at[idx_d.at[jm]], add=True)
            if with_deg:
                pltpu.sync_copy(ones_v, accd.at[idx_d.at[jm]], add=True)

        gather(0, rows0, sem0)

        def step(t, carry):
            j0 = 2 * t
            g = j0 // GRP
            jm = j0 % GRP

            @pl.when(jm == 0)
            def _():
                pltpu.sync_copy(dstr.at[wid, pl.ds(g * GRP, GRP)], idx_d)

            gather(j0 + 1, rows1, sem1)
            drain_scatter(jm, rows0, sem0)

            @pl.when(jnp.logical_and(jm == GRP // 4, g + 1 < NGROUP))
            def _():
                def load(par):
                    pltpu.sync_copy(srcr.at[wid, pl.ds((g + 1) * GRP, GRP)],
                                    idx_s[par])

                both_parities((g + 1) % 2, load)

            @pl.when(t + 1 < NCHUNK // 2)
            def _():
                gather(j0 + 2, rows0, sem0)

            drain_scatter(jm + 1, rows1, sem1)
            return carry

        lax.fori_loop(0, NCHUNK // 2, step, 0)
        plsc.subcore_barrier()
        pltpu.sync_copy(acc.at[pl.ds(r0, RPS)], p_out.at[c, pl.ds(r0, RPS)])
        if with_deg:
            pltpu.sync_copy(accd.at[pl.ds(r0, RPS)],
                            degp_out.at[c, pl.ds(r0, RPS)])

    return pl.kernel(body, out_type=out_type, mesh=_mesh,
                     scratch_types=scratch,
                     compiler_params=pltpu.CompilerParams(
                         use_tc_tiling_on_sc=False))


_sc_segsum_deg = _make_sc_segsum(with_deg=True)
_sc_segsum = _make_sc_segsum(with_deg=False)

BN = 1000
_GRID = N // BN


def _tc0_body(x, wr, bl, xr):
    xr[...] = (jnp.dot(x[...], wr[...], preferred_element_type=jnp.float32)
               + bl[...])


def _tc1_body(p, d, xr, wl, hpre, stats):
    i = pl.program_id(0)
    deg = jnp.maximum(d[0, :, 0:1] + d[1, :, 0:1], 1.0)
    agg = (p[0] + p[1]) / deg
    h = jnp.dot(agg, wl[...], preferred_element_type=jnp.float32) + xr[...]
    hpre[...] = h
    ss = jnp.concatenate([jnp.sum(h, 0, keepdims=True),
                          jnp.sum(h * h, 0, keepdims=True)], axis=0)

    @pl.when(i == 0)
    def _():
        stats[...] = ss

    @pl.when(i != 0)
    def _():
        stats[...] = stats[...] + ss


def _bn_relu(hpre, stats, gamma, beta):
    st = stats[...]
    mean = st[0:1, :] * (1.0 / N)
    var = st[1:2, :] * (1.0 / N) - mean * mean
    scale = gamma[...] * lax.rsqrt(var + 1e-5)
    return jnp.maximum((hpre[...] - mean) * scale + beta[...], 0.0)


def _tc2a_body(hpre, stats, gamma, beta, wl2, p2):
    h = _bn_relu(hpre, stats, gamma, beta)
    p2[...] = jnp.dot(h, wl2[...], preferred_element_type=jnp.float32)


def _tc2b_body(hpre, stats, gamma, beta, wr2, bl2, r2b):
    h = _bn_relu(hpre, stats, gamma, beta)
    r2b[...] = (jnp.dot(h, wr2[...], preferred_element_type=jnp.float32)
                + bl2[...])


def _tc3_body(q, d, r2b, out):
    deg = jnp.maximum(d[0, :, 0:1] + d[1, :, 0:1], 1.0)
    out[...] = (q[0] + q[1]) / deg + r2b[...]


def _row_spec(w):
    return pl.BlockSpec((BN, w), lambda i: (i, 0))


def _pair_spec(w):
    return pl.BlockSpec((2, BN, w), lambda i: (0, i, 0))


def _full_spec(shape):
    return pl.BlockSpec(shape, lambda i: tuple(0 for _ in shape))


_tc0 = pl.pallas_call(
    _tc0_body,
    grid=(_GRID,),
    in_specs=[_row_spec(128), _full_spec((128, 256)), _full_spec((1, 256))],
    out_specs=_row_spec(256),
    out_shape=jax.ShapeDtypeStruct((N, 256), jnp.float32),
)

_tc1 = pl.pallas_call(
    _tc1_body,
    grid=(_GRID,),
    in_specs=[_pair_spec(128), _pair_spec(16),
              _row_spec(256), _full_spec((128, 256))],
    out_specs=[_row_spec(256), _full_spec((2, 256))],
    out_shape=[jax.ShapeDtypeStruct((N, 256), jnp.float32),
               jax.ShapeDtypeStruct((2, 256), jnp.float32)],
)

_tc2a = pl.pallas_call(
    _tc2a_body,
    grid=(_GRID,),
    in_specs=[_row_spec(256), _full_spec((2, 256)), _full_spec((1, 256)),
              _full_spec((1, 256)), _full_spec((256, 128))],
    out_specs=_row_spec(128),
    out_shape=jax.ShapeDtypeStruct((N, 128), jnp.float32),
)

_tc2b = pl.pallas_call(
    _tc2b_body,
    grid=(_GRID,),
    in_specs=[_row_spec(256), _full_spec((2, 256)), _full_spec((1, 256)),
              _full_spec((1, 256)), _full_spec((256, 128)),
              _full_spec((1, 128))],
    out_specs=_row_spec(128),
    out_shape=jax.ShapeDtypeStruct((N, 128), jnp.float32),
)

_tc3 = pl.pallas_call(
    _tc3_body,
    grid=(_GRID,),
    in_specs=[_pair_spec(128), _pair_spec(16), _row_spec(128)],
    out_specs=_row_spec(128),
    out_shape=jax.ShapeDtypeStruct((N, 128), jnp.float32),
)


def kernel(x, edge_index, Wl1, bl1, Wr1, Wl2, bl2, Wr2, gamma, beta):
    src = edge_index[0].reshape(NW, NCHUNK, CHUNK)
    dst = edge_index[1].reshape(NW, NCHUNK, CHUNK)
    zrows = jnp.zeros((N, 128), jnp.float32)
    zdeg = jnp.zeros((N, 16), jnp.float32)
    ones_h = jnp.ones((CHUNK, 16), jnp.float32)

    xr = _tc0(x, Wr1, bl1.reshape(1, 256))
    P, degP = _sc_segsum_deg(x, src, dst, zrows, zdeg, ones_h)
    hpre, stats = _tc1(P, degP, xr, Wl1)
    gam = gamma.reshape(1, 256)
    bet = beta.reshape(1, 256)
    p2 = _tc2a(hpre, stats, gam, bet, Wl2)
    r2b = _tc2b(hpre, stats, gam, bet, Wr2, bl2.reshape(1, 128))
    (Q,) = _sc_segsum(p2, src, dst, zrows)
    out = _tc3(Q, degP, r2b)
    return out
